# baseline (speedup 1.0000x reference)
import jax
import jax.numpy as jnp
from jax import lax
from jax.experimental import pallas as pl
from jax.experimental.pallas import tpu as pltpu
from jax.experimental.pallas import tpu_sc as plsc

_RES = 512
_FEAT = 48
_NPTS = 1048576

_NC = 2
_NS = 16
_NW = _NC * _NS
_PPW = _NPTS // _NW
_C = 128
_NCHUNK = _PPW // _C
_NPB = _NPTS // _C
_NFB = _FEAT // 8
_OSTRIDE = _C + 8


def _body(xy, tab, out,
          xy0, i0, w0, r0, o0,
          xy1, i1, w1, r1, o1,
          sg0, sg1, so0, so1):
    cid = lax.axis_index("c")
    sid = lax.axis_index("s")
    wid = sid * _NC + cid
    pb0 = wid * _NCHUNK

    bufs = ((xy0, i0, w0, r0, o0, sg0, so0),
            (xy1, i1, w1, r1, o1, sg1, so1))

    def stage_a(c, xy_v, i_v, w_v, r_v, sg):
        pltpu.sync_copy(xy.at[pb0 + c], xy_v)

        def grp(g, carry):
            off = g * 16
            sl = pl.ds(off, 16)
            x = xy_v[0, sl] * float(_RES - 1)
            xi = x.astype(jnp.int32)
            xi = jnp.minimum(jnp.maximum(xi, 0), _RES - 2)
            wx = x - xi.astype(jnp.float32)
            y = xy_v[1, sl] * float(_RES - 1)
            yi = y.astype(jnp.int32)
            yi = jnp.minimum(jnp.maximum(yi, 0), _RES - 2)
            wy = y - yi.astype(jnp.float32)
            b = xi * _RES + yi
            i_v[0, sl] = b
            i_v[1, sl] = b + 1
            i_v[2, sl] = b + _RES
            i_v[3, sl] = b + (_RES + 1)
            u = 1.0 - wx
            v = 1.0 - wy
            w_v[0, sl] = u * v
            w_v[1, sl] = u * wy
            w_v[2, sl] = wx * v
            w_v[3, sl] = wx * wy
            return carry

        lax.fori_loop(0, _C // 16, grp, None)
        for t in range(4):
            pltpu.async_copy(tab.at[i_v.at[t]], r_v.at[t], sg)

    def stage_b(c, i_v, w_v, r_v, o_v, sg, so):
        for t in range(4):
            pltpu.make_async_copy(tab.at[i_v.at[t]], r_v.at[t], sg).wait()

        @pl.when(c >= 2)
        def _():
            for f in range(_FEAT):
                pltpu.make_async_copy(
                    o_v.at[f, pl.ds(0, _C)],
                    out.at[f // 8, pb0 + c - 2, pl.ds((f % 8) * _C, _C)],
                    so).wait()

        lane = lax.iota(jnp.int32, 16)

        def grp(g, carry):
            off = g * 16
            sl = pl.ds(off, 16)
            v00 = w_v[0, sl]
            v01 = w_v[1, sl]
            v10 = w_v[2, sl]
            v11 = w_v[3, sl]
            for k in range(16):
                p = off + k
                s00 = v00[k]
                s01 = v01[k]
                s10 = v10[k]
                s11 = v11[k]
                for j in range(_FEAT // 16):
                    fsl = pl.ds(j * 16, 16)
                    acc = (r_v[0, p, fsl] * s00 + r_v[1, p, fsl] * s01
                           + r_v[2, p, fsl] * s10 + r_v[3, p, fsl] * s11)
                    plsc.store_scatter(
                        o_v, [j * 16 + lane, jnp.full((16,), p, jnp.int32)], acc)
            return carry

        lax.fori_loop(0, _C // 16, grp, None)
        for f in range(_FEAT):
            pltpu.async_copy(o_v.at[f, pl.ds(0, _C)],
                             out.at[f // 8, pb0 + c, pl.ds((f % 8) * _C, _C)],
                             so)

    stage_a(0, bufs[0][0], bufs[0][1], bufs[0][2], bufs[0][3], bufs[0][5])

    def pair(ci, carry):
        for u in range(2):
            c = ci * 2 + u
            cur = bufs[u]
            nxt = bufs[1 - u]

            @pl.when(c + 1 < _NCHUNK)
            def _():
                stage_a(c + 1, nxt[0], nxt[1], nxt[2], nxt[3], nxt[5])

            stage_b(c, cur[1], cur[2], cur[3], cur[4], cur[5], cur[6])
        return carry

    lax.fori_loop(0, _NCHUNK // 2, pair, None)

    for u, c in ((0, _NCHUNK - 2), (1, _NCHUNK - 1)):
        for f in range(_FEAT):
            pltpu.make_async_copy(
                bufs[u][4].at[f, pl.ds(0, _C)],
                out.at[f // 8, pb0 + c, pl.ds((f % 8) * _C, _C)],
                bufs[u][6]).wait()


def kernel(pts, codebook0):
    xy = pts.reshape(_NPB, _C, 2).transpose(0, 2, 1)
    mesh = plsc.VectorSubcoreMesh(core_axis_name="c", subcore_axis_name="s")
    buf_set = [
        pltpu.VMEM((2, _C), jnp.float32),
        pltpu.VMEM((4, _C), jnp.int32),
        pltpu.VMEM((4, _C), jnp.float32),
        pltpu.VMEM((4, _C, _FEAT), jnp.float32),
        pltpu.VMEM((_FEAT, _OSTRIDE), jnp.float32),
    ]
    f = pl.kernel(
        _body,
        mesh=mesh,
        compiler_params=pltpu.CompilerParams(
            use_tc_tiling_on_sc=False, needs_layout_passes=False),
        out_type=jax.ShapeDtypeStruct((_NFB, _NPB, 8 * _C), jnp.float32),
        scratch_types=buf_set + buf_set + [
            pltpu.SemaphoreType.DMA,
            pltpu.SemaphoreType.DMA,
            pltpu.SemaphoreType.DMA,
            pltpu.SemaphoreType.DMA,
        ],
    )
    o = f(xy, codebook0)
    return (o.reshape(_NFB, _NPB, 8, _C)
             .transpose(1, 3, 0, 2)
             .reshape(_NPTS, _FEAT))


# speedup vs baseline: 1.0554x; 1.0554x over previous
import jax
import jax.numpy as jnp
from jax import lax
from jax.experimental import pallas as pl
from jax.experimental.pallas import tpu as pltpu
from jax.experimental.pallas import tpu_sc as plsc

_RES = 512
_FEAT = 48
_NPTS = 1048576

_NC = 2
_NS = 16
_NW = _NC * _NS
_PPW = _NPTS // _NW
_C = 128
_NCHUNK = _PPW // _C
_NPB = _NPTS // _C
_NFB = _FEAT // 8
_OSTRIDE = _C + 8


def _body(xy, tab, out,
          xy0, i0, w0, r0, o0,
          xy1, i1, w1, r1, o1,
          sg0, sg1, so0, so1):
    cid = lax.axis_index("c")
    sid = lax.axis_index("s")
    wid = sid * _NC + cid
    pb0 = wid * _NCHUNK

    bufs = ((xy0, i0, w0, r0, o0, sg0, so0),
            (xy1, i1, w1, r1, o1, sg1, so1))

    def stage_a(c, xy_v, i_v, w_v, r_v, sg):
        pltpu.sync_copy(xy.at[pb0 + c], xy_v)

        def grp(g, carry):
            off = g * 16
            sl = pl.ds(off, 16)
            x = xy_v[0, sl] * float(_RES - 1)
            xi = x.astype(jnp.int32)
            xi = jnp.minimum(jnp.maximum(xi, 0), _RES - 2)
            wx = x - xi.astype(jnp.float32)
            y = xy_v[1, sl] * float(_RES - 1)
            yi = y.astype(jnp.int32)
            yi = jnp.minimum(jnp.maximum(yi, 0), _RES - 2)
            wy = y - yi.astype(jnp.float32)
            b = xi * _RES + yi
            i_v[0, sl] = b
            i_v[1, sl] = b + 1
            i_v[2, sl] = b + _RES
            i_v[3, sl] = b + (_RES + 1)
            u = 1.0 - wx
            v = 1.0 - wy
            w_v[0, sl] = u * v
            w_v[1, sl] = u * wy
            w_v[2, sl] = wx * v
            w_v[3, sl] = wx * wy
            return carry

        lax.fori_loop(0, _C // 16, grp, None)
        for t in range(4):
            pltpu.async_copy(tab.at[i_v.at[t]], r_v.at[t], sg)

    def stage_b(c, i_v, w_v, r_v, o_v, sg, so):
        for t in range(4):
            pltpu.make_async_copy(tab.at[i_v.at[t]], r_v.at[t], sg).wait()

        @pl.when(c >= 2)
        def _():
            for fb in range(_NFB):
                pltpu.make_async_copy(
                    o_v.at[pl.ds(fb * 8, 8), pl.ds(0, _C)],
                    out.at[fb, pb0 + c - 2], so).wait()

        lane = lax.iota(jnp.int32, 16)

        def grp(g, carry):
            off = g * 16
            sl = pl.ds(off, 16)
            v00 = w_v[0, sl]
            v01 = w_v[1, sl]
            v10 = w_v[2, sl]
            v11 = w_v[3, sl]
            for k in range(16):
                p = off + k
                s00 = v00[k]
                s01 = v01[k]
                s10 = v10[k]
                s11 = v11[k]
                for j in range(_FEAT // 16):
                    fsl = pl.ds(j * 16, 16)
                    acc = (r_v[0, p, fsl] * s00 + r_v[1, p, fsl] * s01
                           + r_v[2, p, fsl] * s10 + r_v[3, p, fsl] * s11)
                    plsc.store_scatter(
                        o_v, [j * 16 + lane, jnp.full((16,), p, jnp.int32)], acc)
            return carry

        lax.fori_loop(0, _C // 16, grp, None)
        for fb in range(_NFB):
            pltpu.async_copy(o_v.at[pl.ds(fb * 8, 8), pl.ds(0, _C)],
                             out.at[fb, pb0 + c], so)

    stage_a(0, bufs[0][0], bufs[0][1], bufs[0][2], bufs[0][3], bufs[0][5])

    def pair(ci, carry):
        for u in range(2):
            c = ci * 2 + u
            cur = bufs[u]
            nxt = bufs[1 - u]

            @pl.when(c + 1 < _NCHUNK)
            def _():
                stage_a(c + 1, nxt[0], nxt[1], nxt[2], nxt[3], nxt[5])

            stage_b(c, cur[1], cur[2], cur[3], cur[4], cur[5], cur[6])
        return carry

    lax.fori_loop(0, _NCHUNK // 2, pair, None)

    for u, c in ((0, _NCHUNK - 2), (1, _NCHUNK - 1)):
        for fb in range(_NFB):
            pltpu.make_async_copy(
                bufs[u][4].at[pl.ds(fb * 8, 8), pl.ds(0, _C)],
                out.at[fb, pb0 + c], bufs[u][6]).wait()


def kernel(pts, codebook0):
    xy = pts.reshape(_NPB, _C, 2).transpose(0, 2, 1)
    mesh = plsc.VectorSubcoreMesh(core_axis_name="c", subcore_axis_name="s")
    buf_set = [
        pltpu.VMEM((2, _C), jnp.float32),
        pltpu.VMEM((4, _C), jnp.int32),
        pltpu.VMEM((4, _C), jnp.float32),
        pltpu.VMEM((4, _C, _FEAT), jnp.float32),
        pltpu.VMEM((_FEAT, _OSTRIDE), jnp.float32),
    ]
    f = pl.kernel(
        _body,
        mesh=mesh,
        compiler_params=pltpu.CompilerParams(
            use_tc_tiling_on_sc=False, needs_layout_passes=False),
        out_type=jax.ShapeDtypeStruct((_NFB, _NPB, 8, _C), jnp.float32),
        scratch_types=buf_set + buf_set + [
            pltpu.SemaphoreType.DMA,
            pltpu.SemaphoreType.DMA,
            pltpu.SemaphoreType.DMA,
            pltpu.SemaphoreType.DMA,
        ],
    )
    o = f(xy, codebook0)
    return o.transpose(1, 3, 0, 2).reshape(_NPTS, _FEAT)
